# Initial kernel scaffold; baseline (speedup 1.0000x reference)
import jax
import jax.numpy as jnp
from jax.experimental import pallas as pl


def kernel(pred_char_struc, pred_sc_logits, pred_lr_compo_logits, pred_ul_compo_logits):
    raise NotImplementedError("write your pallas kernel here")


# speedup vs baseline: 2.0479x; 2.0479x over previous
import jax
import jax.numpy as jnp
from jax import lax
from jax.experimental import pallas as pl
from jax.experimental.pallas import tpu as pltpu
from jax.experimental.pallas import tpu_sc as plsc

B = 1024
K = 10
S = 10
VSC = 10000
VCO = 1000
L = 16
NCORES = 2
NSUBCORES = 16
NWORKERS = NCORES * NSUBCORES
ROWS_PER_W = B // NWORKERS
NEG = float("-inf")
BIGI = 1 << 20

SC_G, SC_C = 25, 25
CO_G, CO_C = 9, 7


def _lane():
    return lax.iota(jnp.int32, L)


def _dg(x, idx):
    return lax.gather(
        x, idx[:, None],
        lax.GatherDimensionNumbers(
            offset_dims=(), collapsed_slice_dims=(0,), start_index_map=(0,)),
        (1,), mode=lax.GatherScatterMode.PROMISE_IN_BOUNDS)


def _splat(x, dtype=jnp.int32):
    return jnp.full((L,), x, dtype=dtype)


def _bfly(v, op):
    lane = _lane()
    for j in (8, 4, 2, 1):
        v = op(v, _dg(v, lane ^ j))
    return v


def _bmax(v):
    return _bfly(v, jnp.maximum)


def _bsum_scalar(m):
    return _bfly(jnp.where(m, 1, 0), jnp.add)[0]


def _cgt_i(av, ai, bv, bi):
    g1 = jnp.where(av > bv, 1, 0)
    e1 = jnp.where(av == bv, 1, 0)
    l1 = jnp.where(ai < bi, 1, 0)
    return g1 + e1 * l1


def _sort16(v, i):
    lane = _lane()
    for k in (2, 4, 8, 16):
        j = k // 2
        while j >= 1:
            pidx = lane ^ j
            pv = _dg(v, pidx)
            pi = _dg(i, pidx)
            gi = _cgt_i(v, i, pv, pi)
            bk = lax.rem(lax.div(lane, _splat(k)), _splat(2))
            bj = lax.rem(lax.div(lane, _splat(j)), _splat(2))
            wm = 1 - (bk + bj - 2 * bk * bj)
            sw = gi + wm
            v = jnp.where(sw == 1, pv, v)
            i = jnp.where(sw == 1, pi, i)
            j //= 2
    return v, i


def _merge16(av, ai, bv, bi):
    lane = _lane()
    rev = _splat(15) - lane
    rbv = _dg(bv, rev)
    rbi = _dg(bi, rev)
    gi = _cgt_i(av, ai, rbv, rbi)
    hv = jnp.where(gi > 0, av, rbv)
    hi = jnp.where(gi > 0, ai, rbi)
    for j in (8, 4, 2, 1):
        pidx = lane ^ j
        pv = _dg(hv, pidx)
        pi = _dg(hi, pidx)
        gi = _cgt_i(hv, hi, pv, pi)
        lower = 1 - lax.rem(lax.div(lane, _splat(j)), _splat(2))
        sw = gi + lower
        hv = jnp.where(sw == 1, pv, hv)
        hi = jnp.where(sw == 1, pi, hi)
    return hv, hi


def _scan_groups(data_ref, base, ngroups, nchunks, nvalid, gbuf):
    lane = _lane()

    def g_body(g, run):
        def c_body(cc, gm):
            t = g * nchunks + cc
            v = data_ref[pl.ds(base + t * L, L)]
            if nvalid % L != 0:
                v = jnp.where(t * L + lane < nvalid, v, NEG)
            return jnp.maximum(gm, v)

        gm = lax.fori_loop(0, nchunks, c_body, _splat(NEG, jnp.float32))
        gbuf[pl.ds(g * L, L)] = gm
        return jnp.maximum(run, gm)

    return lax.fori_loop(0, ngroups, g_body, _splat(NEG, jnp.float32))


def _top10(data_ref, gbuf, best_v, best_i):
    lane = _lane()
    run = _scan_groups(data_ref, 0, SC_G, SC_C, VSC, gbuf)
    srt, _ = _sort16(run, lane)
    tau = _dg(srt, _splat(K - 1))

    best_v[pl.ds(0, L)] = _splat(NEG, jnp.float32)
    best_i[pl.ds(0, L)] = _splat(BIGI) + lane

    def g_body(g, _):
        gm = gbuf[pl.ds(g * L, L)]
        hit = _bsum_scalar(gm >= tau)

        @pl.when(hit > 0)
        def _():
            def c_body(cc, _):
                t = g * SC_C + cc
                v = data_ref[pl.ds(t * L, L)]
                chit = _bsum_scalar(v >= tau)

                @pl.when(chit > 0)
                def _():
                    sv, si = _sort16(v, t * L + lane)
                    mv, mi = _merge16(best_v[pl.ds(0, L)],
                                      best_i[pl.ds(0, L)], sv, si)
                    best_v[pl.ds(0, L)] = mv
                    best_i[pl.ds(0, L)] = mi

                return 0

            lax.fori_loop(0, SC_C, c_body, 0)

        return 0

    lax.fori_loop(0, SC_G, g_body, 0)
    return jnp.where(lane < K, best_i[pl.ds(0, L)], 0)


def _branch_seq(score_ref, seq_ref, gbuf, best_v, best_i):
    lane = _lane()

    def step_body(t, _):
        base = t * VCO
        run = _scan_groups(score_ref, base, CO_G, CO_C, VCO, gbuf)
        msplat = _bmax(run)

        best_v[pl.ds(0, L)] = _splat(NEG, jnp.float32)
        best_i[pl.ds(0, L)] = lane

        def g_body(g, _):
            gm = gbuf[pl.ds(g * L, L)]
            hit = _bsum_scalar(gm == msplat)

            @pl.when(hit > 0)
            def _():
                def c_body(cc, _):
                    tt = g * CO_C + cc
                    v = score_ref[pl.ds(base + tt * L, L)]
                    mi = jnp.where(v == msplat, 1, 0) * jnp.where(
                        tt * L + lane < VCO, 1, 0)
                    chit = _bfly(mi, jnp.add)[0]

                    @pl.when(chit > 0)
                    def _():
                        mi2 = jnp.where(v == msplat, 1, 0) * jnp.where(
                            tt * L + lane < VCO, 1, 0)
                        negidx = jnp.where(
                            mi2 > 0, -(tt * L + lane), -BIGI
                        ).astype(jnp.float32)
                        sv, si = _sort16(negidx, lane)
                        mv, mi = _merge16(best_v[pl.ds(0, L)],
                                          best_i[pl.ds(0, L)], sv, si)
                        best_v[pl.ds(0, L)] = mv
                        best_i[pl.ds(0, L)] = mi

                    return 0

                lax.fori_loop(0, CO_C, c_body, 0)

            return 0

        lax.fori_loop(0, CO_G, g_body, 0)

        bv = best_v[pl.ds(0, L)]
        tied = (-bv).astype(jnp.int32)
        tcnt = _bsum_scalar(bv > float(-BIGI))
        tsplat = _splat(0) + tcnt
        c = lax.rem(lane, tsplat)
        sym = _dg(tied, c)

        @pl.when(tcnt > 1)
        def _():
            r = lax.div(lane, tsplat)

            def pos_body(p, _):
                rowv = seq_ref[pl.ds(p * L, L)]
                seq_ref[pl.ds(p * L, L)] = _dg(rowv, r)
                return 0

            lax.fori_loop(0, t, pos_body, 0)

        seq_ref[pl.ds(t * L, L)] = sym
        return 0

    lax.fori_loop(0, S, step_body, 0)


def _sc_body(struc_hbm, sc_hbm, lr_hbm, ul_hbm,
             out_sc, out_lr, out_ul, out_res,
             in_sc, in_lr, in_ul, gbuf, best_v, best_i,
             seq_lr, seq_ul, struc_v, st_sc, st_lr, st_ul, st_res):
    wid = lax.axis_index("s") * NCORES + lax.axis_index("c")
    lane = _lane()

    pltpu.sync_copy(struc_hbm.at[pl.ds(wid * ROWS_PER_W * L,
                                       ROWS_PER_W * L)], struc_v)

    def row_body(rl, _):
        row = wid * ROWS_PER_W + rl
        pltpu.sync_copy(sc_hbm.at[pl.ds(row * VSC, VSC)], in_sc)
        pltpu.sync_copy(lr_hbm.at[pl.ds(row * S * VCO, S * VCO)],
                        in_lr.at[pl.ds(0, S * VCO)])
        pltpu.sync_copy(ul_hbm.at[pl.ds(row * S * VCO, S * VCO)],
                        in_ul.at[pl.ds(0, S * VCO)])

        sc_i = _top10(in_sc, gbuf, best_v, best_i)
        _branch_seq(in_lr, seq_lr, gbuf, best_v, best_i)
        _branch_seq(in_ul, seq_ul, gbuf, best_v, best_i)

        s = struc_v[pl.ds(rl * L, L)]
        obase = rl * S * L

        def asm_body(p, _):
            keep = _lane() < K
            lr_row = jnp.where(keep, seq_lr[pl.ds(p * L, L)], 0)
            ul_row = jnp.where(keep, seq_ul[pl.ds(p * L, L)], 0)
            pz = jnp.where(_splat(0) + p == 0, 1, 0) * jnp.where(
                _lane() < K, 1, 0)
            sc_first = jnp.where(pz > 0, sc_i, _splat(0))
            res = jnp.where(s == 1, lr_row,
                            jnp.where(s == 2, ul_row, sc_first))
            st_lr[pl.ds(obase + p * L, L)] = lr_row
            st_ul[pl.ds(obase + p * L, L)] = ul_row
            st_res[pl.ds(obase + p * L, L)] = res
            return 0

        lax.fori_loop(0, S, asm_body, 0)
        st_sc[pl.ds(rl * L, L)] = jnp.where(_lane() < K, sc_i, 0)
        return 0

    lax.fori_loop(0, ROWS_PER_W, row_body, 0)

    pltpu.sync_copy(st_sc, out_sc.at[pl.ds(wid * ROWS_PER_W * L,
                                           ROWS_PER_W * L)])
    pltpu.sync_copy(st_lr, out_lr.at[pl.ds(wid * ROWS_PER_W * S * L,
                                           ROWS_PER_W * S * L)])
    pltpu.sync_copy(st_ul, out_ul.at[pl.ds(wid * ROWS_PER_W * S * L,
                                           ROWS_PER_W * S * L)])
    pltpu.sync_copy(st_res, out_res.at[pl.ds(wid * ROWS_PER_W * S * L,
                                             ROWS_PER_W * S * L)])


@jax.jit
def _run(struc, sc_flat, lr_flat, ul_flat):
    mesh = plsc.VectorSubcoreMesh(core_axis_name="c", subcore_axis_name="s",
                                  num_cores=NCORES, num_subcores=NSUBCORES)
    f = pl.kernel(
        _sc_body,
        out_type=[
            jax.ShapeDtypeStruct((B * L,), jnp.int32),
            jax.ShapeDtypeStruct((B * S * L,), jnp.int32),
            jax.ShapeDtypeStruct((B * S * L,), jnp.int32),
            jax.ShapeDtypeStruct((B * S * L,), jnp.int32),
        ],
        mesh=mesh,
        scratch_types=[
            pltpu.VMEM((VSC,), jnp.float32),
            pltpu.VMEM((S * VCO + L,), jnp.float32),
            pltpu.VMEM((S * VCO + L,), jnp.float32),
            pltpu.VMEM((SC_G * L,), jnp.float32),
            pltpu.VMEM((L,), jnp.float32),
            pltpu.VMEM((L,), jnp.int32),
            pltpu.VMEM((S * L,), jnp.int32),
            pltpu.VMEM((S * L,), jnp.int32),
            pltpu.VMEM((ROWS_PER_W * L,), jnp.int32),
            pltpu.VMEM((ROWS_PER_W * L,), jnp.int32),
            pltpu.VMEM((ROWS_PER_W * S * L,), jnp.int32),
            pltpu.VMEM((ROWS_PER_W * S * L,), jnp.int32),
            pltpu.VMEM((ROWS_PER_W * S * L,), jnp.int32),
        ],
    )
    return f(struc, sc_flat, lr_flat, ul_flat)


def kernel(pred_char_struc, pred_sc_logits, pred_lr_compo_logits,
           pred_ul_compo_logits):
    lr_scores = jnp.log(jax.nn.softmax(pred_lr_compo_logits, axis=-1))
    ul_scores = jnp.log(jax.nn.softmax(pred_ul_compo_logits, axis=-1))
    struc_x = jnp.repeat(pred_char_struc, L)
    o_sc, o_lr, o_ul, o_res = _run(
        struc_x, pred_sc_logits.reshape(-1),
        lr_scores.reshape(-1), ul_scores.reshape(-1))
    o_sc = o_sc.reshape(B, L)[:, :K]
    o_lr = o_lr.reshape(B, S, L).transpose(0, 2, 1)[:, :K, :]
    o_ul = o_ul.reshape(B, S, L).transpose(0, 2, 1)[:, :K, :]
    o_res = o_res.reshape(B, S, L).transpose(0, 2, 1)[:, :K, :]
    return (o_sc, o_lr, o_ul, o_res)


# speedup vs baseline: 2.1839x; 1.0665x over previous
import jax
import jax.numpy as jnp
from jax import lax
from jax.experimental import pallas as pl
from jax.experimental.pallas import tpu as pltpu
from jax.experimental.pallas import tpu_sc as plsc

B = 1024
K = 10
S = 10
VSC = 10000
VCO = 1000
L = 16
NCORES = 2
NSUBCORES = 16
NWORKERS = NCORES * NSUBCORES
ROWS_PER_W = B // NWORKERS
NEG = float("-inf")
BIGI = 1 << 20

SC_G, SC_C = 25, 25
CO_G, CO_C = 9, 7


def _lane():
    return lax.iota(jnp.int32, L)


def _dg(x, idx):
    return lax.gather(
        x, idx[:, None],
        lax.GatherDimensionNumbers(
            offset_dims=(), collapsed_slice_dims=(0,), start_index_map=(0,)),
        (1,), mode=lax.GatherScatterMode.PROMISE_IN_BOUNDS)


def _splat(x, dtype=jnp.int32):
    return jnp.full((L,), x, dtype=dtype)


def _bfly(v, op):
    lane = _lane()
    for j in (8, 4, 2, 1):
        v = op(v, _dg(v, lane ^ j))
    return v


def _bmax(v):
    return _bfly(v, jnp.maximum)


def _bsum_scalar(m):
    return _bfly(jnp.where(m, 1, 0), jnp.add)[0]


def _cgt_i(av, ai, bv, bi):
    g1 = jnp.where(av > bv, 1, 0)
    e1 = jnp.where(av == bv, 1, 0)
    l1 = jnp.where(ai < bi, 1, 0)
    return g1 + e1 * l1


def _sort16(v, i):
    lane = _lane()
    for k in (2, 4, 8, 16):
        j = k // 2
        while j >= 1:
            pidx = lane ^ j
            pv = _dg(v, pidx)
            pi = _dg(i, pidx)
            gi = _cgt_i(v, i, pv, pi)
            bk = lax.rem(lax.div(lane, _splat(k)), _splat(2))
            bj = lax.rem(lax.div(lane, _splat(j)), _splat(2))
            wm = 1 - (bk + bj - 2 * bk * bj)
            sw = gi + wm
            v = jnp.where(sw == 1, pv, v)
            i = jnp.where(sw == 1, pi, i)
            j //= 2
    return v, i


def _merge16(av, ai, bv, bi):
    lane = _lane()
    rev = _splat(15) - lane
    rbv = _dg(bv, rev)
    rbi = _dg(bi, rev)
    gi = _cgt_i(av, ai, rbv, rbi)
    hv = jnp.where(gi > 0, av, rbv)
    hi = jnp.where(gi > 0, ai, rbi)
    for j in (8, 4, 2, 1):
        pidx = lane ^ j
        pv = _dg(hv, pidx)
        pi = _dg(hi, pidx)
        gi = _cgt_i(hv, hi, pv, pi)
        lower = 1 - lax.rem(lax.div(lane, _splat(j)), _splat(2))
        sw = gi + lower
        hv = jnp.where(sw == 1, pv, hv)
        hi = jnp.where(sw == 1, pi, hi)
    return hv, hi


def _scan_groups(data_ref, base, ngroups, nchunks, nvalid, gbuf):
    lane = _lane()

    def g_body(g, run):
        def c_body(cc, gm):
            t = g * nchunks + cc
            v = data_ref[pl.ds(base + t * L, L)]
            if nvalid % L != 0:
                v = jnp.where(t * L + lane < nvalid, v, NEG)
            return jnp.maximum(gm, v)

        gm = lax.fori_loop(0, nchunks, c_body, _splat(NEG, jnp.float32))
        gbuf[pl.ds(g * L, L)] = gm
        return jnp.maximum(run, gm)

    return lax.fori_loop(0, ngroups, g_body, _splat(NEG, jnp.float32))


def _top10(data_ref, sbase, gbuf, best_v, best_i):
    lane = _lane()
    run = _scan_groups(data_ref, sbase, SC_G, SC_C, VSC, gbuf)
    srt, _ = _sort16(run, lane)
    tau = _dg(srt, _splat(K - 1))

    best_v[pl.ds(0, L)] = _splat(NEG, jnp.float32)
    best_i[pl.ds(0, L)] = _splat(BIGI) + lane

    def g_body(g, _):
        gm = gbuf[pl.ds(g * L, L)]
        hit = _bsum_scalar(gm >= tau)

        @pl.when(hit > 0)
        def _():
            def c_body(cc, _):
                t = g * SC_C + cc
                v = data_ref[pl.ds(sbase + t * L, L)]
                chit = _bsum_scalar(v >= tau)

                @pl.when(chit > 0)
                def _():
                    sv, si = _sort16(v, t * L + lane)
                    mv, mi = _merge16(best_v[pl.ds(0, L)],
                                      best_i[pl.ds(0, L)], sv, si)
                    best_v[pl.ds(0, L)] = mv
                    best_i[pl.ds(0, L)] = mi

                return 0

            lax.fori_loop(0, SC_C, c_body, 0)

        return 0

    lax.fori_loop(0, SC_G, g_body, 0)
    return jnp.where(lane < K, best_i[pl.ds(0, L)], 0)


def _branch_seq(score_ref, sbase, seq_ref, gbuf, best_v, best_i):
    lane = _lane()

    def step_body(t, _):
        base = sbase + t * VCO
        run = _scan_groups(score_ref, base, CO_G, CO_C, VCO, gbuf)
        msplat = _bmax(run)

        best_v[pl.ds(0, L)] = _splat(NEG, jnp.float32)
        best_i[pl.ds(0, L)] = lane

        def g_body(g, _):
            gm = gbuf[pl.ds(g * L, L)]
            hit = _bsum_scalar(gm == msplat)

            @pl.when(hit > 0)
            def _():
                def c_body(cc, _):
                    tt = g * CO_C + cc
                    v = score_ref[pl.ds(base + tt * L, L)]
                    mi = jnp.where(v == msplat, 1, 0) * jnp.where(
                        tt * L + lane < VCO, 1, 0)
                    chit = _bfly(mi, jnp.add)[0]

                    @pl.when(chit > 0)
                    def _():
                        mi2 = jnp.where(v == msplat, 1, 0) * jnp.where(
                            tt * L + lane < VCO, 1, 0)
                        negidx = jnp.where(
                            mi2 > 0, -(tt * L + lane), -BIGI
                        ).astype(jnp.float32)
                        sv, si = _sort16(negidx, lane)
                        mv, mi = _merge16(best_v[pl.ds(0, L)],
                                          best_i[pl.ds(0, L)], sv, si)
                        best_v[pl.ds(0, L)] = mv
                        best_i[pl.ds(0, L)] = mi

                    return 0

                lax.fori_loop(0, CO_C, c_body, 0)

            return 0

        lax.fori_loop(0, CO_G, g_body, 0)

        bv = best_v[pl.ds(0, L)]
        tied = (-bv).astype(jnp.int32)
        tcnt = _bsum_scalar(bv > float(-BIGI))
        tsplat = _splat(0) + tcnt
        c = lax.rem(lane, tsplat)
        sym = _dg(tied, c)

        @pl.when(tcnt > 1)
        def _():
            r = lax.div(lane, tsplat)

            def pos_body(p, _):
                rowv = seq_ref[pl.ds(p * L, L)]
                seq_ref[pl.ds(p * L, L)] = _dg(rowv, r)
                return 0

            lax.fori_loop(0, t, pos_body, 0)

        seq_ref[pl.ds(t * L, L)] = sym
        return 0

    lax.fori_loop(0, S, step_body, 0)


def _sc_body(struc_hbm, sc_hbm, lr_hbm, ul_hbm,
             out_sc, out_lr, out_ul, out_res,
             in_sc, in_lr, in_ul, gbuf, best_v, best_i,
             seq_lr, seq_ul, struc_v, st_sc, st_lr, st_ul, st_res,
             sem_sc, sem_lr, sem_ul):
    wid = lax.axis_index("s") * NCORES + lax.axis_index("c")
    lane = _lane()

    pltpu.sync_copy(struc_hbm.at[pl.ds(wid * ROWS_PER_W * L,
                                       ROWS_PER_W * L)], struc_v)

    def _dma(rl, half, start):
        row = wid * ROWS_PER_W + rl
        cps = (
            pltpu.make_async_copy(sc_hbm.at[pl.ds(row * VSC, VSC)],
                                  in_sc.at[pl.ds(half * VSC, VSC)], sem_sc),
            pltpu.make_async_copy(lr_hbm.at[pl.ds(row * S * VCO, S * VCO)],
                                  in_lr.at[pl.ds(half * S * VCO, S * VCO)],
                                  sem_lr),
            pltpu.make_async_copy(ul_hbm.at[pl.ds(row * S * VCO, S * VCO)],
                                  in_ul.at[pl.ds(half * S * VCO, S * VCO)],
                                  sem_ul),
        )
        for cp in cps:
            if start:
                cp.start()
            else:
                cp.wait()

    _dma(0, 0, True)

    def row_body(rl, _):
        half = lax.rem(rl, 2)

        @pl.when(rl + 1 < ROWS_PER_W)
        def _():
            _dma(rl + 1, 1 - half, True)

        _dma(rl, half, False)

        sc_i = _top10(in_sc, half * VSC, gbuf, best_v, best_i)
        _branch_seq(in_lr, half * S * VCO, seq_lr, gbuf, best_v, best_i)
        _branch_seq(in_ul, half * S * VCO, seq_ul, gbuf, best_v, best_i)

        s = struc_v[pl.ds(rl * L, L)]
        obase = rl * S * L

        def asm_body(p, _):
            keep = _lane() < K
            lr_row = jnp.where(keep, seq_lr[pl.ds(p * L, L)], 0)
            ul_row = jnp.where(keep, seq_ul[pl.ds(p * L, L)], 0)
            pz = jnp.where(_splat(0) + p == 0, 1, 0) * jnp.where(
                _lane() < K, 1, 0)
            sc_first = jnp.where(pz > 0, sc_i, _splat(0))
            res = jnp.where(s == 1, lr_row,
                            jnp.where(s == 2, ul_row, sc_first))
            st_lr[pl.ds(obase + p * L, L)] = lr_row
            st_ul[pl.ds(obase + p * L, L)] = ul_row
            st_res[pl.ds(obase + p * L, L)] = res
            return 0

        lax.fori_loop(0, S, asm_body, 0)
        st_sc[pl.ds(rl * L, L)] = jnp.where(_lane() < K, sc_i, 0)
        return 0

    lax.fori_loop(0, ROWS_PER_W, row_body, 0)

    pltpu.sync_copy(st_sc, out_sc.at[pl.ds(wid * ROWS_PER_W * L,
                                           ROWS_PER_W * L)])
    pltpu.sync_copy(st_lr, out_lr.at[pl.ds(wid * ROWS_PER_W * S * L,
                                           ROWS_PER_W * S * L)])
    pltpu.sync_copy(st_ul, out_ul.at[pl.ds(wid * ROWS_PER_W * S * L,
                                           ROWS_PER_W * S * L)])
    pltpu.sync_copy(st_res, out_res.at[pl.ds(wid * ROWS_PER_W * S * L,
                                             ROWS_PER_W * S * L)])


@jax.jit
def _run(struc, sc_flat, lr_flat, ul_flat):
    mesh = plsc.VectorSubcoreMesh(core_axis_name="c", subcore_axis_name="s",
                                  num_cores=NCORES, num_subcores=NSUBCORES)
    f = pl.kernel(
        _sc_body,
        out_type=[
            jax.ShapeDtypeStruct((B * L,), jnp.int32),
            jax.ShapeDtypeStruct((B * S * L,), jnp.int32),
            jax.ShapeDtypeStruct((B * S * L,), jnp.int32),
            jax.ShapeDtypeStruct((B * S * L,), jnp.int32),
        ],
        mesh=mesh,
        scratch_types=[
            pltpu.VMEM((2 * VSC,), jnp.float32),
            pltpu.VMEM((2 * S * VCO + L,), jnp.float32),
            pltpu.VMEM((2 * S * VCO + L,), jnp.float32),
            pltpu.VMEM((SC_G * L,), jnp.float32),
            pltpu.VMEM((L,), jnp.float32),
            pltpu.VMEM((L,), jnp.int32),
            pltpu.VMEM((S * L,), jnp.int32),
            pltpu.VMEM((S * L,), jnp.int32),
            pltpu.VMEM((ROWS_PER_W * L,), jnp.int32),
            pltpu.VMEM((ROWS_PER_W * L,), jnp.int32),
            pltpu.VMEM((ROWS_PER_W * S * L,), jnp.int32),
            pltpu.VMEM((ROWS_PER_W * S * L,), jnp.int32),
            pltpu.VMEM((ROWS_PER_W * S * L,), jnp.int32),
            pltpu.SemaphoreType.DMA,
            pltpu.SemaphoreType.DMA,
            pltpu.SemaphoreType.DMA,
        ],
    )
    return f(struc, sc_flat, lr_flat, ul_flat)


def kernel(pred_char_struc, pred_sc_logits, pred_lr_compo_logits,
           pred_ul_compo_logits):
    lr_scores = jnp.log(jax.nn.softmax(pred_lr_compo_logits, axis=-1))
    ul_scores = jnp.log(jax.nn.softmax(pred_ul_compo_logits, axis=-1))
    struc_x = jnp.repeat(pred_char_struc, L)
    o_sc, o_lr, o_ul, o_res = _run(
        struc_x, pred_sc_logits.reshape(-1),
        lr_scores.reshape(-1), ul_scores.reshape(-1))
    o_sc = o_sc.reshape(B, L)[:, :K]
    o_lr = o_lr.reshape(B, S, L).transpose(0, 2, 1)[:, :K, :]
    o_ul = o_ul.reshape(B, S, L).transpose(0, 2, 1)[:, :K, :]
    o_res = o_res.reshape(B, S, L).transpose(0, 2, 1)[:, :K, :]
    return (o_sc, o_lr, o_ul, o_res)


# speedup vs baseline: 2.4387x; 1.1167x over previous
import jax
import jax.numpy as jnp
from jax import lax
from jax.experimental import pallas as pl
from jax.experimental.pallas import tpu as pltpu
from jax.experimental.pallas import tpu_sc as plsc

B = 1024
K = 10
S = 10
VSC = 10000
VCO = 1000
L = 16
NCORES = 2
NSUBCORES = 16
NWORKERS = NCORES * NSUBCORES
ROWS_PER_W = B // NWORKERS
NEG = float("-inf")
BIGI = 1 << 20

SC_G, SC_C = 25, 25
CO_G, CO_C = 9, 7


def _lane():
    return lax.iota(jnp.int32, L)


def _dg(x, idx):
    return lax.gather(
        x, idx[:, None],
        lax.GatherDimensionNumbers(
            offset_dims=(), collapsed_slice_dims=(0,), start_index_map=(0,)),
        (1,), mode=lax.GatherScatterMode.PROMISE_IN_BOUNDS)


def _splat(x, dtype=jnp.int32):
    return jnp.full((L,), x, dtype=dtype)


def _bfly(v, op):
    lane = _lane()
    for j in (8, 4, 2, 1):
        v = op(v, _dg(v, lane ^ j))
    return v


def _bmax(v):
    return _bfly(v, jnp.maximum)


def _bsum_scalar(m):
    return _bfly(jnp.where(m, 1, 0), jnp.add)[0]


def _cgt_i(av, ai, bv, bi):
    g1 = jnp.where(av > bv, 1, 0)
    e1 = jnp.where(av == bv, 1, 0)
    l1 = jnp.where(ai < bi, 1, 0)
    return g1 + e1 * l1


def _sort16(v, i):
    lane = _lane()
    for k in (2, 4, 8, 16):
        j = k // 2
        while j >= 1:
            pidx = lane ^ j
            pv = _dg(v, pidx)
            pi = _dg(i, pidx)
            gi = _cgt_i(v, i, pv, pi)
            bk = lax.rem(lax.div(lane, _splat(k)), _splat(2))
            bj = lax.rem(lax.div(lane, _splat(j)), _splat(2))
            wm = 1 - (bk + bj - 2 * bk * bj)
            sw = gi + wm
            v = jnp.where(sw == 1, pv, v)
            i = jnp.where(sw == 1, pi, i)
            j //= 2
    return v, i


def _merge16(av, ai, bv, bi):
    lane = _lane()
    rev = _splat(15) - lane
    rbv = _dg(bv, rev)
    rbi = _dg(bi, rev)
    gi = _cgt_i(av, ai, rbv, rbi)
    hv = jnp.where(gi > 0, av, rbv)
    hi = jnp.where(gi > 0, ai, rbi)
    for j in (8, 4, 2, 1):
        pidx = lane ^ j
        pv = _dg(hv, pidx)
        pi = _dg(hi, pidx)
        gi = _cgt_i(hv, hi, pv, pi)
        lower = 1 - lax.rem(lax.div(lane, _splat(j)), _splat(2))
        sw = gi + lower
        hv = jnp.where(sw == 1, pv, hv)
        hi = jnp.where(sw == 1, pi, hi)
    return hv, hi


def _scan_groups(data_ref, base, ngroups, nchunks, nvalid, gbuf):
    lane = _lane()

    def g_body(g, run):
        gm = _splat(NEG, jnp.float32)
        for cc in range(nchunks):
            t = g * nchunks + cc
            v = data_ref[pl.ds(base + t * L, L)]
            if nvalid % L != 0:
                v = jnp.where(t * L + lane < nvalid, v, NEG)
            gm = jnp.maximum(gm, v)

        gbuf[pl.ds(g * L, L)] = gm
        return jnp.maximum(run, gm)

    return lax.fori_loop(0, ngroups, g_body, _splat(NEG, jnp.float32))


def _top10(data_ref, sbase, gbuf, best_v, best_i):
    lane = _lane()
    run = _scan_groups(data_ref, sbase, SC_G, SC_C, VSC, gbuf)
    srt, _ = _sort16(run, lane)
    tau = _dg(srt, _splat(K - 1))

    best_v[pl.ds(0, L)] = _splat(NEG, jnp.float32)
    best_i[pl.ds(0, L)] = _splat(BIGI) + lane

    def g_body(g, _):
        gm = gbuf[pl.ds(g * L, L)]
        hit = _bsum_scalar(gm >= tau)

        @pl.when(hit > 0)
        def _():
            def c_body(cc, _):
                t = g * SC_C + cc
                v = data_ref[pl.ds(sbase + t * L, L)]
                chit = _bsum_scalar(v >= tau)

                @pl.when(chit > 0)
                def _():
                    sv, si = _sort16(v, t * L + lane)
                    mv, mi = _merge16(best_v[pl.ds(0, L)],
                                      best_i[pl.ds(0, L)], sv, si)
                    best_v[pl.ds(0, L)] = mv
                    best_i[pl.ds(0, L)] = mi

                return 0

            lax.fori_loop(0, SC_C, c_body, 0)

        return 0

    lax.fori_loop(0, SC_G, g_body, 0)
    return jnp.where(lane < K, best_i[pl.ds(0, L)], 0)


def _branch_seq(score_ref, sbase, seq_ref, gbuf, best_v, best_i):
    lane = _lane()

    def step_body(t, _):
        base = sbase + t * VCO
        run = _scan_groups(score_ref, base, CO_G, CO_C, VCO, gbuf)
        msplat = _bmax(run)

        best_v[pl.ds(0, L)] = _splat(NEG, jnp.float32)
        best_i[pl.ds(0, L)] = lane

        def g_body(g, _):
            gm = gbuf[pl.ds(g * L, L)]
            hit = _bsum_scalar(gm == msplat)

            @pl.when(hit > 0)
            def _():
                def c_body(cc, _):
                    tt = g * CO_C + cc
                    v = score_ref[pl.ds(base + tt * L, L)]
                    mi = jnp.where(v == msplat, 1, 0) * jnp.where(
                        tt * L + lane < VCO, 1, 0)
                    chit = _bfly(mi, jnp.add)[0]

                    @pl.when(chit > 0)
                    def _():
                        mi2 = jnp.where(v == msplat, 1, 0) * jnp.where(
                            tt * L + lane < VCO, 1, 0)
                        negidx = jnp.where(
                            mi2 > 0, -(tt * L + lane), -BIGI
                        ).astype(jnp.float32)
                        sv, si = _sort16(negidx, lane)
                        mv, mi = _merge16(best_v[pl.ds(0, L)],
                                          best_i[pl.ds(0, L)], sv, si)
                        best_v[pl.ds(0, L)] = mv
                        best_i[pl.ds(0, L)] = mi

                    return 0

                lax.fori_loop(0, CO_C, c_body, 0)

            return 0

        lax.fori_loop(0, CO_G, g_body, 0)

        bv = best_v[pl.ds(0, L)]
        tied = (-bv).astype(jnp.int32)
        tcnt = _bsum_scalar(bv > float(-BIGI))
        tsplat = _splat(0) + tcnt
        c = lax.rem(lane, tsplat)
        sym = _dg(tied, c)

        @pl.when(tcnt > 1)
        def _():
            r = lax.div(lane, tsplat)

            def pos_body(p, _):
                rowv = seq_ref[pl.ds(p * L, L)]
                seq_ref[pl.ds(p * L, L)] = _dg(rowv, r)
                return 0

            lax.fori_loop(0, t, pos_body, 0)

        seq_ref[pl.ds(t * L, L)] = sym
        return 0

    lax.fori_loop(0, S, step_body, 0)


def _sc_body(struc_hbm, sc_hbm, lr_hbm, ul_hbm,
             out_sc, out_lr, out_ul, out_res,
             in_sc, in_lr, in_ul, gbuf, best_v, best_i,
             seq_lr, seq_ul, struc_v, st_sc, st_lr, st_ul, st_res,
             sem_sc, sem_lr, sem_ul):
    wid = lax.axis_index("s") * NCORES + lax.axis_index("c")
    lane = _lane()

    pltpu.sync_copy(struc_hbm.at[pl.ds(wid * ROWS_PER_W * L,
                                       ROWS_PER_W * L)], struc_v)

    def _dma(rl, half, start):
        row = wid * ROWS_PER_W + rl
        cps = (
            pltpu.make_async_copy(sc_hbm.at[pl.ds(row * VSC, VSC)],
                                  in_sc.at[pl.ds(half * VSC, VSC)], sem_sc),
            pltpu.make_async_copy(lr_hbm.at[pl.ds(row * S * VCO, S * VCO)],
                                  in_lr.at[pl.ds(half * S * VCO, S * VCO)],
                                  sem_lr),
            pltpu.make_async_copy(ul_hbm.at[pl.ds(row * S * VCO, S * VCO)],
                                  in_ul.at[pl.ds(half * S * VCO, S * VCO)],
                                  sem_ul),
        )
        for cp in cps:
            if start:
                cp.start()
            else:
                cp.wait()

    _dma(0, 0, True)

    def row_body(rl, _):
        half = lax.rem(rl, 2)

        @pl.when(rl + 1 < ROWS_PER_W)
        def _():
            _dma(rl + 1, 1 - half, True)

        _dma(rl, half, False)

        sc_i = _top10(in_sc, half * VSC, gbuf, best_v, best_i)
        _branch_seq(in_lr, half * S * VCO, seq_lr, gbuf, best_v, best_i)
        _branch_seq(in_ul, half * S * VCO, seq_ul, gbuf, best_v, best_i)

        s = struc_v[pl.ds(rl * L, L)]
        obase = rl * S * L

        def asm_body(p, _):
            keep = _lane() < K
            lr_row = jnp.where(keep, seq_lr[pl.ds(p * L, L)], 0)
            ul_row = jnp.where(keep, seq_ul[pl.ds(p * L, L)], 0)
            pz = jnp.where(_splat(0) + p == 0, 1, 0) * jnp.where(
                _lane() < K, 1, 0)
            sc_first = jnp.where(pz > 0, sc_i, _splat(0))
            res = jnp.where(s == 1, lr_row,
                            jnp.where(s == 2, ul_row, sc_first))
            st_lr[pl.ds(obase + p * L, L)] = lr_row
            st_ul[pl.ds(obase + p * L, L)] = ul_row
            st_res[pl.ds(obase + p * L, L)] = res
            return 0

        lax.fori_loop(0, S, asm_body, 0)
        st_sc[pl.ds(rl * L, L)] = jnp.where(_lane() < K, sc_i, 0)
        return 0

    lax.fori_loop(0, ROWS_PER_W, row_body, 0)

    pltpu.sync_copy(st_sc, out_sc.at[pl.ds(wid * ROWS_PER_W * L,
                                           ROWS_PER_W * L)])
    pltpu.sync_copy(st_lr, out_lr.at[pl.ds(wid * ROWS_PER_W * S * L,
                                           ROWS_PER_W * S * L)])
    pltpu.sync_copy(st_ul, out_ul.at[pl.ds(wid * ROWS_PER_W * S * L,
                                           ROWS_PER_W * S * L)])
    pltpu.sync_copy(st_res, out_res.at[pl.ds(wid * ROWS_PER_W * S * L,
                                             ROWS_PER_W * S * L)])


@jax.jit
def _run(struc, sc_flat, lr_flat, ul_flat):
    mesh = plsc.VectorSubcoreMesh(core_axis_name="c", subcore_axis_name="s",
                                  num_cores=NCORES, num_subcores=NSUBCORES)
    f = pl.kernel(
        _sc_body,
        out_type=[
            jax.ShapeDtypeStruct((B * L,), jnp.int32),
            jax.ShapeDtypeStruct((B * S * L,), jnp.int32),
            jax.ShapeDtypeStruct((B * S * L,), jnp.int32),
            jax.ShapeDtypeStruct((B * S * L,), jnp.int32),
        ],
        mesh=mesh,
        scratch_types=[
            pltpu.VMEM((2 * VSC,), jnp.float32),
            pltpu.VMEM((2 * S * VCO + L,), jnp.float32),
            pltpu.VMEM((2 * S * VCO + L,), jnp.float32),
            pltpu.VMEM((SC_G * L,), jnp.float32),
            pltpu.VMEM((L,), jnp.float32),
            pltpu.VMEM((L,), jnp.int32),
            pltpu.VMEM((S * L,), jnp.int32),
            pltpu.VMEM((S * L,), jnp.int32),
            pltpu.VMEM((ROWS_PER_W * L,), jnp.int32),
            pltpu.VMEM((ROWS_PER_W * L,), jnp.int32),
            pltpu.VMEM((ROWS_PER_W * S * L,), jnp.int32),
            pltpu.VMEM((ROWS_PER_W * S * L,), jnp.int32),
            pltpu.VMEM((ROWS_PER_W * S * L,), jnp.int32),
            pltpu.SemaphoreType.DMA,
            pltpu.SemaphoreType.DMA,
            pltpu.SemaphoreType.DMA,
        ],
    )
    return f(struc, sc_flat, lr_flat, ul_flat)


def kernel(pred_char_struc, pred_sc_logits, pred_lr_compo_logits,
           pred_ul_compo_logits):
    lr_scores = jnp.log(jax.nn.softmax(pred_lr_compo_logits, axis=-1))
    ul_scores = jnp.log(jax.nn.softmax(pred_ul_compo_logits, axis=-1))
    struc_x = jnp.repeat(pred_char_struc, L)
    o_sc, o_lr, o_ul, o_res = _run(
        struc_x, pred_sc_logits.reshape(-1),
        lr_scores.reshape(-1), ul_scores.reshape(-1))
    o_sc = o_sc.reshape(B, L)[:, :K]
    o_lr = o_lr.reshape(B, S, L).transpose(0, 2, 1)[:, :K, :]
    o_ul = o_ul.reshape(B, S, L).transpose(0, 2, 1)[:, :K, :]
    o_res = o_res.reshape(B, S, L).transpose(0, 2, 1)[:, :K, :]
    return (o_sc, o_lr, o_ul, o_res)


# speedup vs baseline: 3.4053x; 1.3963x over previous
import jax
import jax.numpy as jnp
from jax import lax
from jax.experimental import pallas as pl
from jax.experimental.pallas import tpu as pltpu
from jax.experimental.pallas import tpu_sc as plsc

B = 1024
K = 10
S = 10
VSC = 10000
VCO = 1000
L = 16
NCORES = 2
NSUBCORES = 16
NWORKERS = NCORES * NSUBCORES
ROWS_PER_W = B // NWORKERS
NEG = float("-inf")
BIGI = 1 << 20

SC_G, SC_C = 25, 25
CO_G, CO_C = 9, 7


def _lane():
    return lax.iota(jnp.int32, L)


def _dg(x, idx):
    return lax.gather(
        x, idx[:, None],
        lax.GatherDimensionNumbers(
            offset_dims=(), collapsed_slice_dims=(0,), start_index_map=(0,)),
        (1,), mode=lax.GatherScatterMode.PROMISE_IN_BOUNDS)


def _splat(x, dtype=jnp.int32):
    return jnp.full((L,), x, dtype=dtype)


def _bfly(v, op):
    lane = _lane()
    for j in (8, 4, 2, 1):
        v = op(v, _dg(v, lane ^ j))
    return v


def _bmax(v):
    return _bfly(v, jnp.maximum)


def _bsum_scalar(m):
    return _bfly(jnp.where(m, 1, 0), jnp.add)[0]


def _cgt_i(av, ai, bv, bi):
    g1 = jnp.where(av > bv, 1, 0)
    e1 = jnp.where(av == bv, 1, 0)
    l1 = jnp.where(ai < bi, 1, 0)
    return g1 + e1 * l1


def _sort16(v, i):
    lane = _lane()
    for k in (2, 4, 8, 16):
        j = k // 2
        while j >= 1:
            pidx = lane ^ j
            pv = _dg(v, pidx)
            pi = _dg(i, pidx)
            gi = _cgt_i(v, i, pv, pi)
            bk = lax.rem(lax.div(lane, _splat(k)), _splat(2))
            bj = lax.rem(lax.div(lane, _splat(j)), _splat(2))
            wm = 1 - (bk + bj - 2 * bk * bj)
            sw = gi + wm
            v = jnp.where(sw == 1, pv, v)
            i = jnp.where(sw == 1, pi, i)
            j //= 2
    return v, i


def _merge16(av, ai, bv, bi):
    lane = _lane()
    rev = _splat(15) - lane
    rbv = _dg(bv, rev)
    rbi = _dg(bi, rev)
    gi = _cgt_i(av, ai, rbv, rbi)
    hv = jnp.where(gi > 0, av, rbv)
    hi = jnp.where(gi > 0, ai, rbi)
    for j in (8, 4, 2, 1):
        pidx = lane ^ j
        pv = _dg(hv, pidx)
        pi = _dg(hi, pidx)
        gi = _cgt_i(hv, hi, pv, pi)
        lower = 1 - lax.rem(lax.div(lane, _splat(j)), _splat(2))
        sw = gi + lower
        hv = jnp.where(sw == 1, pv, hv)
        hi = jnp.where(sw == 1, pi, hi)
    return hv, hi


def _scan_groups(data_ref, base, ngroups, nchunks, nvalid, gbuf):
    lane = _lane()

    def g_body(g, run):
        gm = _splat(NEG, jnp.float32)
        for cc in range(nchunks):
            t = g * nchunks + cc
            v = data_ref[pl.ds(base + t * L, L)]
            if nvalid % L != 0:
                v = jnp.where(t * L + lane < nvalid, v, NEG)
            gm = jnp.maximum(gm, v)

        gbuf[pl.ds(g * L, L)] = gm
        return jnp.maximum(run, gm)

    return lax.fori_loop(0, ngroups, g_body, _splat(NEG, jnp.float32))


def _top10(data_ref, sbase, gbuf, best_v, best_i):
    lane = _lane()
    run = _scan_groups(data_ref, sbase, SC_G, SC_C, VSC, gbuf)
    srt, _ = _sort16(run, lane)
    tau = _dg(srt, _splat(K - 1))

    best_v[pl.ds(0, L)] = _splat(NEG, jnp.float32)
    best_i[pl.ds(0, L)] = _splat(BIGI) + lane

    def g_body(g, _):
        gm = gbuf[pl.ds(g * L, L)]
        hit = _bsum_scalar(gm >= tau)

        @pl.when(hit > 0)
        def _():
            def c_body(cc, _):
                t = g * SC_C + cc
                v = data_ref[pl.ds(sbase + t * L, L)]
                chit = _bsum_scalar(v >= tau)

                @pl.when(chit > 0)
                def _():
                    sv, si = _sort16(v, t * L + lane)
                    mv, mi = _merge16(best_v[pl.ds(0, L)],
                                      best_i[pl.ds(0, L)], sv, si)
                    best_v[pl.ds(0, L)] = mv
                    best_i[pl.ds(0, L)] = mi

                return 0

            lax.fori_loop(0, SC_C, c_body, 0)

        return 0

    lax.fori_loop(0, SC_G, g_body, 0)
    return jnp.where(lane < K, best_i[pl.ds(0, L)], 0)


def _branch_seq(score_ref, sbase, seq_ref, best_v, best_i):
    lane = _lane()
    nch = (VCO + L - 1) // L

    def step_body(t, _):
        base = sbase + t * VCO
        run = _splat(NEG, jnp.float32)
        cnt = _splat(0)
        rj = _splat(0)
        for cc in range(nch):
            v = score_ref[pl.ds(base + cc * L, L)]
            if (cc + 1) * L > VCO:
                v = jnp.where(cc * L + lane < VCO, v, NEG)
            gt = jnp.where(v > run, 1, 0)
            eq = jnp.where(v == run, 1, 0)
            cnt = jnp.where(gt > 0, 1, cnt + eq)
            rj = jnp.where(gt > 0, _splat(cc), rj)
            run = jnp.maximum(run, v)

        msplat = _bmax(run)
        meq = jnp.where(run == msplat, 1, 0)
        tsplat = _bfly(meq * cnt, jnp.add)
        amin = _bfly(jnp.where(meq > 0, rj * L + lane, _splat(BIGI)),
                     jnp.minimum)
        seq_ref[pl.ds(t * L, L)] = amin
        tcnt = tsplat[0]

        @pl.when(tcnt > 1)
        def _():
            best_v[pl.ds(0, L)] = _splat(NEG, jnp.float32)
            best_i[pl.ds(0, L)] = lane

            def c_body(tt, _):
                v = score_ref[pl.ds(base + tt * L, L)]
                mi = jnp.where(v == msplat, 1, 0) * jnp.where(
                    tt * L + lane < VCO, 1, 0)
                chit = _bfly(mi, jnp.add)[0]

                @pl.when(chit > 0)
                def _():
                    mi2 = jnp.where(v == msplat, 1, 0) * jnp.where(
                        tt * L + lane < VCO, 1, 0)
                    negidx = jnp.where(
                        mi2 > 0, -(tt * L + lane), -BIGI).astype(jnp.float32)
                    sv, si = _sort16(negidx, lane)
                    mv, mi3 = _merge16(best_v[pl.ds(0, L)],
                                       best_i[pl.ds(0, L)], sv, si)
                    best_v[pl.ds(0, L)] = mv
                    best_i[pl.ds(0, L)] = mi3

                return 0

            lax.fori_loop(0, nch, c_body, 0)
            tied = (-best_v[pl.ds(0, L)]).astype(jnp.int32)
            tv = jnp.full((L,), tcnt, jnp.int32)
            c = lax.rem(lane, tv)
            r = lax.div(lane, tv)
            seq_ref[pl.ds(t * L, L)] = _dg(tied, c)

            def pos_body(p, _):
                rowv = seq_ref[pl.ds(p * L, L)]
                seq_ref[pl.ds(p * L, L)] = _dg(rowv, r)
                return 0

            lax.fori_loop(0, t, pos_body, 0)

        return 0

    lax.fori_loop(0, S, step_body, 0)


def _sc_body(struc_hbm, sc_hbm, lr_hbm, ul_hbm,
             out_sc, out_lr, out_ul, out_res,
             in_sc, in_lr, in_ul, gbuf, best_v, best_i,
             seq_lr, seq_ul, struc_v, st_sc, st_lr, st_ul, st_res,
             sem_sc, sem_lr, sem_ul):
    wid = lax.axis_index("s") * NCORES + lax.axis_index("c")
    lane = _lane()

    pltpu.sync_copy(struc_hbm.at[pl.ds(wid * ROWS_PER_W * L,
                                       ROWS_PER_W * L)], struc_v)

    def _dma(rl, half, start):
        row = wid * ROWS_PER_W + rl
        cps = (
            pltpu.make_async_copy(sc_hbm.at[pl.ds(row * VSC, VSC)],
                                  in_sc.at[pl.ds(half * VSC, VSC)], sem_sc),
            pltpu.make_async_copy(lr_hbm.at[pl.ds(row * S * VCO, S * VCO)],
                                  in_lr.at[pl.ds(half * S * VCO, S * VCO)],
                                  sem_lr),
            pltpu.make_async_copy(ul_hbm.at[pl.ds(row * S * VCO, S * VCO)],
                                  in_ul.at[pl.ds(half * S * VCO, S * VCO)],
                                  sem_ul),
        )
        for cp in cps:
            if start:
                cp.start()
            else:
                cp.wait()

    _dma(0, 0, True)

    def row_body(rl, _):
        half = lax.rem(rl, 2)

        @pl.when(rl + 1 < ROWS_PER_W)
        def _():
            _dma(rl + 1, 1 - half, True)

        _dma(rl, half, False)

        sc_i = _top10(in_sc, half * VSC, gbuf, best_v, best_i)
        _branch_seq(in_lr, half * S * VCO, seq_lr, best_v, best_i)
        _branch_seq(in_ul, half * S * VCO, seq_ul, best_v, best_i)

        s = struc_v[pl.ds(rl * L, L)]
        obase = rl * S * L

        def asm_body(p, _):
            keep = _lane() < K
            lr_row = jnp.where(keep, seq_lr[pl.ds(p * L, L)], 0)
            ul_row = jnp.where(keep, seq_ul[pl.ds(p * L, L)], 0)
            pz = jnp.where(_splat(0) + p == 0, 1, 0) * jnp.where(
                _lane() < K, 1, 0)
            sc_first = jnp.where(pz > 0, sc_i, _splat(0))
            res = jnp.where(s == 1, lr_row,
                            jnp.where(s == 2, ul_row, sc_first))
            st_lr[pl.ds(obase + p * L, L)] = lr_row
            st_ul[pl.ds(obase + p * L, L)] = ul_row
            st_res[pl.ds(obase + p * L, L)] = res
            return 0

        lax.fori_loop(0, S, asm_body, 0)
        st_sc[pl.ds(rl * L, L)] = jnp.where(_lane() < K, sc_i, 0)
        return 0

    lax.fori_loop(0, ROWS_PER_W, row_body, 0)

    pltpu.sync_copy(st_sc, out_sc.at[pl.ds(wid * ROWS_PER_W * L,
                                           ROWS_PER_W * L)])
    pltpu.sync_copy(st_lr, out_lr.at[pl.ds(wid * ROWS_PER_W * S * L,
                                           ROWS_PER_W * S * L)])
    pltpu.sync_copy(st_ul, out_ul.at[pl.ds(wid * ROWS_PER_W * S * L,
                                           ROWS_PER_W * S * L)])
    pltpu.sync_copy(st_res, out_res.at[pl.ds(wid * ROWS_PER_W * S * L,
                                             ROWS_PER_W * S * L)])


@jax.jit
def _run(struc, sc_flat, lr_flat, ul_flat):
    mesh = plsc.VectorSubcoreMesh(core_axis_name="c", subcore_axis_name="s",
                                  num_cores=NCORES, num_subcores=NSUBCORES)
    f = pl.kernel(
        _sc_body,
        out_type=[
            jax.ShapeDtypeStruct((B * L,), jnp.int32),
            jax.ShapeDtypeStruct((B * S * L,), jnp.int32),
            jax.ShapeDtypeStruct((B * S * L,), jnp.int32),
            jax.ShapeDtypeStruct((B * S * L,), jnp.int32),
        ],
        mesh=mesh,
        scratch_types=[
            pltpu.VMEM((2 * VSC,), jnp.float32),
            pltpu.VMEM((2 * S * VCO + L,), jnp.float32),
            pltpu.VMEM((2 * S * VCO + L,), jnp.float32),
            pltpu.VMEM((SC_G * L,), jnp.float32),
            pltpu.VMEM((L,), jnp.float32),
            pltpu.VMEM((L,), jnp.int32),
            pltpu.VMEM((S * L,), jnp.int32),
            pltpu.VMEM((S * L,), jnp.int32),
            pltpu.VMEM((ROWS_PER_W * L,), jnp.int32),
            pltpu.VMEM((ROWS_PER_W * L,), jnp.int32),
            pltpu.VMEM((ROWS_PER_W * S * L,), jnp.int32),
            pltpu.VMEM((ROWS_PER_W * S * L,), jnp.int32),
            pltpu.VMEM((ROWS_PER_W * S * L,), jnp.int32),
            pltpu.SemaphoreType.DMA,
            pltpu.SemaphoreType.DMA,
            pltpu.SemaphoreType.DMA,
        ],
    )
    return f(struc, sc_flat, lr_flat, ul_flat)


def kernel(pred_char_struc, pred_sc_logits, pred_lr_compo_logits,
           pred_ul_compo_logits):
    lr_scores = jnp.log(jax.nn.softmax(pred_lr_compo_logits, axis=-1))
    ul_scores = jnp.log(jax.nn.softmax(pred_ul_compo_logits, axis=-1))
    struc_x = jnp.repeat(pred_char_struc, L)
    o_sc, o_lr, o_ul, o_res = _run(
        struc_x, pred_sc_logits.reshape(-1),
        lr_scores.reshape(-1), ul_scores.reshape(-1))
    o_sc = o_sc.reshape(B, L)[:, :K]
    o_lr = o_lr.reshape(B, S, L).transpose(0, 2, 1)[:, :K, :]
    o_ul = o_ul.reshape(B, S, L).transpose(0, 2, 1)[:, :K, :]
    o_res = o_res.reshape(B, S, L).transpose(0, 2, 1)[:, :K, :]
    return (o_sc, o_lr, o_ul, o_res)
